# baseline (speedup 1.0000x reference)
import functools

import jax
import jax.numpy as jnp
from jax import lax
from jax.experimental import pallas as pl
from jax.experimental.pallas import tpu as pltpu
from jax.experimental.pallas import tpu_sc as plsc

NUM_NODES = 1000000
EMBED_DIM = 64
BATCH = 16384
N_FIELDS = 26
_FPAD = 32
_IDS_PAD = 128
_PADW = 128

_NW = 32
_RPW = BATCH // _NW
_IPW = _RPW * _FPAD
_CLEN = 128
_NCH = _IPW // _CLEN
_NBUF = 4
_NGRP = _NCH // _NBUF


_TBLK = 512


def _make_transpose_pad():
    nblk = (NUM_NODES + _TBLK - 1) // _TBLK

    def body(w_ref, o_ref):
        row = lax.broadcasted_iota(jnp.int32, (EMBED_DIM, EMBED_DIM), 0)
        col = lax.broadcasted_iota(jnp.int32, (EMBED_DIM, EMBED_DIM), 1)
        eye = jnp.where(row == col, 1.0, 0.0).astype(jnp.float32)
        t = lax.dot_general(
            w_ref[...], eye, (((0,), (0,)), ((), ())),
            preferred_element_type=jnp.float32)
        o_ref[...] = jnp.concatenate([t, t], axis=1)

    return pl.pallas_call(
        body,
        grid=(nblk,),
        in_specs=[pl.BlockSpec((EMBED_DIM, _TBLK), lambda i: (0, i))],
        out_specs=pl.BlockSpec((_TBLK, _PADW), lambda i: (i, 0)),
        out_shape=jax.ShapeDtypeStruct((NUM_NODES, _PADW), jnp.float32),
    )


def _make_kernel():
    mesh = plsc.VectorSubcoreMesh(core_axis_name="c", subcore_axis_name="s")

    @functools.partial(
        pl.kernel,
        mesh=mesh,
        compiler_params=pltpu.CompilerParams(use_tc_tiling_on_sc=False),
        out_type=jax.ShapeDtypeStruct((BATCH * _FPAD, _PADW), jnp.float32),
        scratch_types=[
            pltpu.VMEM((_RPW, _FPAD), jnp.int32),
            pltpu.VMEM((_IPW,), jnp.int32),
            pltpu.VMEM((_NBUF, _CLEN, _PADW), jnp.float32),
            pltpu.SemaphoreType.DMA,
            pltpu.SemaphoreType.DMA,
            pltpu.SemaphoreType.DMA,
            pltpu.SemaphoreType.DMA,
            pltpu.SemaphoreType.DMA,
            pltpu.SemaphoreType.DMA,
            pltpu.SemaphoreType.DMA,
            pltpu.SemaphoreType.DMA,
        ],
    )
    def gather_kernel(ids_hbm, table_hbm, out_hbm, idx32_v, idx_v, rows_v,
                      sg0, sg1, sg2, sg3, so0, so1, so2, so3):
        sem_g = [sg0, sg1, sg2, sg3]
        sem_o = [so0, so1, so2, so3]
        wid = lax.axis_index("s") * 2 + lax.axis_index("c")
        base = wid * _RPW
        obase = wid * _IPW

        pltpu.sync_copy(
            ids_hbm.at[pl.ds(base, _RPW), pl.ds(0, _FPAD)], idx32_v)

        def flatten(r, carry):
            idx_v[pl.ds(r * _FPAD, 16)] = idx32_v[r, pl.ds(0, 16)]
            idx_v[pl.ds(r * _FPAD + 16, 16)] = idx32_v[r, pl.ds(16, 16)]
            return carry

        lax.fori_loop(0, _RPW, flatten, 0)

        def start_gather(c, b):
            pltpu.async_copy(
                table_hbm.at[idx_v.at[pl.ds(c * _CLEN, _CLEN)]],
                rows_v.at[b], sem_g[b])

        def wait_gather(b):
            pltpu.make_async_copy(
                table_hbm.at[idx_v.at[pl.ds(0, _CLEN)]],
                rows_v.at[b], sem_g[b]).wait()

        def start_out(c, b):
            pltpu.async_copy(
                rows_v.at[b],
                out_hbm.at[pl.ds(obase + c * _CLEN, _CLEN)], sem_o[b])

        def wait_out(c, b):
            pltpu.make_async_copy(
                rows_v.at[b],
                out_hbm.at[pl.ds(obase + c * _CLEN, _CLEN)], sem_o[b]).wait()

        for b in range(_NBUF):
            start_gather(b, b)

        def body(k, carry):
            for b in range(_NBUF):
                c = _NBUF * k + b
                wait_gather(b)
                start_out(c, b)

                @pl.when(k < _NGRP - 1)
                def _():
                    wait_out(c, b)
                    start_gather(c + _NBUF, b)

            return carry

        lax.fori_loop(0, _NGRP, body, 0)

        for b in range(_NBUF):
            wait_out(_NCH - _NBUF + b, b)

    return gather_kernel


_gather = _make_kernel()
_transpose_pad = _make_transpose_pad()


def kernel(ids, weight):
    ids32 = ids.astype(jnp.int32)
    dummy = (
        lax.broadcasted_iota(jnp.int32, (BATCH, _FPAD - N_FIELDS), 0) * 7
        + lax.broadcasted_iota(jnp.int32, (BATCH, _FPAD - N_FIELDS), 1)
    ) % NUM_NODES
    ids_p = jnp.pad(
        jnp.concatenate([ids32, dummy], axis=1),
        ((0, 0), (0, _IDS_PAD - _FPAD)))
    weight_p = _transpose_pad(weight.T)
    out = _gather(ids_p, weight_p)
    return out.reshape(BATCH, _FPAD, _PADW)[:, :N_FIELDS, :EMBED_DIM]


# speedup vs baseline: 1.9715x; 1.9715x over previous
import functools

import jax
import jax.numpy as jnp
from jax import lax
from jax.experimental import pallas as pl
from jax.experimental.pallas import tpu as pltpu
from jax.experimental.pallas import tpu_sc as plsc

NUM_NODES = 1000000
EMBED_DIM = 64
BATCH = 16384
N_FIELDS = 26
_FPAD = 32
_IDS_PAD = 128
_PADW = 128

_NW = 32
_RPW = BATCH // _NW
_IPW = _RPW * _FPAD
_CLEN = 128
_NCH = _IPW // _CLEN
_NBUF = 4
_NGRP = _NCH // _NBUF


def _make_kernel():
    mesh = plsc.VectorSubcoreMesh(core_axis_name="c", subcore_axis_name="s")

    @functools.partial(
        pl.kernel,
        mesh=mesh,
        compiler_params=pltpu.CompilerParams(use_tc_tiling_on_sc=False),
        out_type=jax.ShapeDtypeStruct((BATCH * _FPAD, _PADW), jnp.float32),
        scratch_types=[
            pltpu.VMEM((_RPW, _FPAD), jnp.int32),
            pltpu.VMEM((_IPW,), jnp.int32),
            pltpu.VMEM((_NBUF, _CLEN, _PADW), jnp.float32),
            pltpu.SemaphoreType.DMA,
            pltpu.SemaphoreType.DMA,
            pltpu.SemaphoreType.DMA,
            pltpu.SemaphoreType.DMA,
            pltpu.SemaphoreType.DMA,
            pltpu.SemaphoreType.DMA,
            pltpu.SemaphoreType.DMA,
            pltpu.SemaphoreType.DMA,
        ],
    )
    def gather_kernel(ids_hbm, table_hbm, out_hbm, idx32_v, idx_v, rows_v,
                      sg0, sg1, sg2, sg3, so0, so1, so2, so3):
        sem_g = [sg0, sg1, sg2, sg3]
        sem_o = [so0, so1, so2, so3]
        wid = lax.axis_index("s") * 2 + lax.axis_index("c")
        base = wid * _RPW
        obase = wid * _IPW

        pltpu.sync_copy(
            ids_hbm.at[pl.ds(base, _RPW), pl.ds(0, _FPAD)], idx32_v)

        def flatten(r, carry):
            idx_v[pl.ds(r * _FPAD, 16)] = idx32_v[r, pl.ds(0, 16)]
            idx_v[pl.ds(r * _FPAD + 16, 16)] = idx32_v[r, pl.ds(16, 16)]
            return carry

        lax.fori_loop(0, _RPW, flatten, 0)

        def start_gather(c, b):
            pltpu.async_copy(
                table_hbm.at[idx_v.at[pl.ds(c * _CLEN, _CLEN)]],
                rows_v.at[b], sem_g[b])

        def wait_gather(b):
            pltpu.make_async_copy(
                table_hbm.at[idx_v.at[pl.ds(0, _CLEN)]],
                rows_v.at[b], sem_g[b]).wait()

        def start_out(c, b):
            pltpu.async_copy(
                rows_v.at[b],
                out_hbm.at[pl.ds(obase + c * _CLEN, _CLEN)], sem_o[b])

        def wait_out(c, b):
            pltpu.make_async_copy(
                rows_v.at[b],
                out_hbm.at[pl.ds(obase + c * _CLEN, _CLEN)], sem_o[b]).wait()

        for b in range(_NBUF):
            start_gather(b, b)

        def body(k, carry):
            for b in range(_NBUF):
                c = _NBUF * k + b
                wait_gather(b)
                start_out(c, b)

                @pl.when(k < _NGRP - 1)
                def _():
                    wait_out(c, b)
                    start_gather(c + _NBUF, b)

            return carry

        lax.fori_loop(0, _NGRP, body, 0)

        for b in range(_NBUF):
            wait_out(_NCH - _NBUF + b, b)

    return gather_kernel


_gather = _make_kernel()


def kernel(ids, weight):
    ids32 = ids.astype(jnp.int32)
    dummy = (
        lax.broadcasted_iota(jnp.int32, (BATCH, _FPAD - N_FIELDS), 0) * 7
        + lax.broadcasted_iota(jnp.int32, (BATCH, _FPAD - N_FIELDS), 1)
    ) % NUM_NODES
    ids_p = jnp.pad(
        jnp.concatenate([ids32, dummy], axis=1),
        ((0, 0), (0, _IDS_PAD - _FPAD)))
    weight_p = jnp.pad(weight, ((0, 0), (0, _PADW - EMBED_DIM)))
    out = _gather(ids_p, weight_p)
    return out.reshape(BATCH, _FPAD, _PADW)[:, :N_FIELDS, :EMBED_DIM]
